# baseline (speedup 1.0000x reference)
import functools

import jax
import jax.numpy as jnp
from jax import lax
from jax.experimental import pallas as pl
from jax.experimental.pallas import tpu as pltpu
from jax.experimental.pallas import tpu_sc as plsc

_N = 10000
_E = 320000
_NFEAT = 128
_NHID = 64
_NCLASS = 40

_NC = 2
_NS = 16
_NT = _NC * _NS
_RPT = _N // _NS
_EPT = _E // _NT
_W = 80
_NWIN = _EPT // _W

_mesh = plsc.VectorSubcoreMesh(core_axis_name="c", subcore_axis_name="s")


_NPAD = 10240
_RPT_PAD = _NPAD // _NS
_LAST = _N - (_NS - 1) * _RPT_PAD
_DEPTH = 16


def _store_out(s, src, dst):
    @pl.when(s < _NS - 1)
    def _full():
        pltpu.sync_copy(src.at[pl.ds(s * _RPT_PAD, _RPT_PAD)],
                        dst.at[pl.ds(s * _RPT_PAD, _RPT_PAD)])

    @pl.when(s == _NS - 1)
    def _clip():
        pltpu.sync_copy(src.at[pl.ds((_NS - 1) * _RPT_PAD, _LAST)],
                        dst.at[pl.ds((_NS - 1) * _RPT_PAD, _LAST)])


@functools.partial(
    pl.kernel,
    out_type=jax.ShapeDtypeStruct((_NC, _NPAD), jnp.float32),
    mesh=_mesh,
    scratch_types=[
        pltpu.VMEM((_NWIN, _W), jnp.int32),
        pltpu.VMEM((_W,), jnp.float32),
        pltpu.VMEM_SHARED((_NPAD,), jnp.float32),
        pltpu.SemaphoreType.DMA,
        pltpu.SemaphoreType.DMA,
    ],
)
def _deg_sc(ei_hbm, zeros_hbm, out_hbm, idxb, ones_v, acc, ssem, isem):
    c = lax.axis_index("c")
    s = lax.axis_index("s")
    tid = c * _NS + s
    one16 = jnp.ones((16,), jnp.float32)
    for i in range(_W // 16):
        ones_v[pl.ds(i * 16, 16)] = one16
    ones_w = ones_v
    iz = pltpu.async_copy(zeros_hbm.at[pl.ds(s * _RPT_PAD, _RPT_PAD)],
                          acc.at[pl.ds(s * _RPT_PAD, _RPT_PAD)], isem)
    ic = pltpu.async_copy(ei_hbm.at[0, tid], idxb, ssem)
    ic.wait()
    iz.wait()
    plsc.subcore_barrier()

    @pl.loop(0, _NWIN)
    def _win(w):
        pltpu.async_copy(ones_w, acc.at[idxb.at[w]], ssem, add=True)

        @pl.when(w >= _DEPTH)
        def _drain():
            pltpu.make_async_copy(ones_w, acc.at[idxb.at[w]], ssem).wait()

    for _ in range(_DEPTH):
        pltpu.make_async_copy(ones_w, acc.at[idxb.at[0]], ssem).wait()
    plsc.subcore_barrier()
    pltpu.sync_copy(acc.at[pl.ds(s * _RPT_PAD, _RPT_PAD)],
                    out_hbm.at[c, pl.ds(s * _RPT_PAD, _RPT_PAD)])


_NB = 5
_NG = _NWIN // _NB


def _make_scatter(d):
    @functools.partial(
        pl.kernel,
        out_type=jax.ShapeDtypeStruct((_NC, _N, d), jnp.float32),
        mesh=_mesh,
        scratch_types=[
            pltpu.VMEM((_NWIN, _W), jnp.int32),
            pltpu.VMEM((_NWIN, _W), jnp.int32),
            pltpu.VMEM((_NB, _W, d), jnp.float32),
            pltpu.VMEM_SHARED((_NPAD, d), jnp.float32),
        ] + [pltpu.SemaphoreType.DMA] * (2 * _NB + 1),
        compiler_params=pltpu.CompilerParams(use_tc_tiling_on_sc=False),
    )
    def _scatter_sc(tab_hbm, ei_hbm, zeros_hbm, out_hbm,
                    ridx, cidx, rows, acc, *sems):
        gsem = sems[:_NB]
        ssem = sems[_NB:2 * _NB]
        isem = sems[2 * _NB]
        c = lax.axis_index("c")
        s = lax.axis_index("s")
        tid = c * _NS + s
        iz = pltpu.async_copy(zeros_hbm.at[pl.ds(s * _RPT_PAD, _RPT_PAD)],
                              acc.at[pl.ds(s * _RPT_PAD, _RPT_PAD)], isem)
        ir = pltpu.async_copy(ei_hbm.at[0, tid], ridx, gsem[0])
        ic = pltpu.async_copy(ei_hbm.at[1, tid], cidx, gsem[1])
        ir.wait()
        ic.wait()
        iz.wait()
        plsc.subcore_barrier()

        for b in range(_NB):
            pltpu.async_copy(tab_hbm.at[ridx.at[b]], rows.at[b], gsem[b])

        @pl.loop(0, _NG)
        def _grp(g):
            w0 = g * _NB
            for b in range(_NB):
                pltpu.make_async_copy(
                    tab_hbm.at[ridx.at[w0 + b]], rows.at[b], gsem[b]).wait()
                pltpu.async_copy(
                    rows.at[b], acc.at[cidx.at[w0 + b]], ssem[b], add=True)

            @pl.when(g + 1 < _NG)
            def _next():
                for b in range(_NB):
                    pltpu.make_async_copy(
                        rows.at[b], acc.at[cidx.at[w0 + b]], ssem[b]).wait()
                    pltpu.async_copy(
                        tab_hbm.at[ridx.at[w0 + _NB + b]], rows.at[b], gsem[b])

        w0 = (_NG - 1) * _NB
        for b in range(_NB):
            pltpu.make_async_copy(
                rows.at[b], acc.at[cidx.at[w0 + b]], ssem[b]).wait()
        plsc.subcore_barrier()
        _store_out(s, acc, out_hbm.at[c])

    return _scatter_sc


_scatter_hid = _make_scatter(_NHID)
_scatter_cls = _make_scatter(_NCLASS)


_RB = 1000
_NRB = _N // _RB


def _tc1_body(dp_ref, x_ref, w_ref, b_ref, h_ref, dis_ref):
    deg = 1.0 + dp_ref[0] + dp_ref[1]
    dis = lax.rsqrt(deg)
    h = jnp.dot(x_ref[...], w_ref[...],
                preferred_element_type=jnp.float32) + b_ref[...]
    h_ref[...] = dis * h
    dis_ref[...] = dis


def _tc2_body(a_ref, h_ref, d_ref, w_ref, b_ref, o_ref):
    dis = d_ref[...]
    u = dis * (a_ref[0] + a_ref[1] + h_ref[...])
    z = jnp.maximum(u, 0.0)
    h2 = jnp.dot(z, w_ref[...],
                 preferred_element_type=jnp.float32) + b_ref[...]
    o_ref[...] = dis * h2


def _tc3_body(a_ref, h_ref, d_ref, o_ref):
    u = d_ref[...] * (a_ref[0] + a_ref[1] + h_ref[...])
    m = jnp.max(u, axis=1, keepdims=True)
    lse = jnp.log(jnp.sum(jnp.exp(u - m), axis=1, keepdims=True)) + m
    o_ref[...] = u - lse


def _col_spec(d):
    return pl.BlockSpec((2, _RB, d), lambda i: (0, i, 0))


def _row_spec(d):
    return pl.BlockSpec((_RB, d), lambda i: (i, 0))


def _full_spec(r, c):
    return pl.BlockSpec((r, c), lambda i: (0, 0))


_tc1 = pl.pallas_call(
    _tc1_body,
    grid=(_NRB,),
    in_specs=[_col_spec(1), _row_spec(_NFEAT),
              _full_spec(_NFEAT, _NHID), _full_spec(1, _NHID)],
    out_specs=(_row_spec(_NHID), _row_spec(1)),
    out_shape=(jax.ShapeDtypeStruct((_N, _NHID), jnp.float32),
               jax.ShapeDtypeStruct((_N, 1), jnp.float32)))
_tc2 = pl.pallas_call(
    _tc2_body,
    grid=(_NRB,),
    in_specs=[_col_spec(_NHID), _row_spec(_NHID), _row_spec(1),
              _full_spec(_NHID, _NCLASS), _full_spec(1, _NCLASS)],
    out_specs=_row_spec(_NCLASS),
    out_shape=jax.ShapeDtypeStruct((_N, _NCLASS), jnp.float32))
_tc3 = pl.pallas_call(
    _tc3_body,
    grid=(_NRB,),
    in_specs=[_col_spec(_NCLASS), _row_spec(_NCLASS), _row_spec(1)],
    out_specs=_row_spec(_NCLASS),
    out_shape=jax.ShapeDtypeStruct((_N, _NCLASS), jnp.float32))


@jax.jit
def kernel(x, edge_index, W1, b1, W2, b2):
    ei4 = edge_index.astype(jnp.int32).reshape(2, _NT, _NWIN, _W)
    zeros_n = jnp.zeros((_NPAD,), jnp.float32)
    zeros_h = jnp.zeros((_NPAD, _NHID), jnp.float32)
    zeros_c = jnp.zeros((_NPAD, _NCLASS), jnp.float32)

    deg_parts = _deg_sc(ei4, zeros_n)[:, :_N, None]
    h1p, dis = _tc1(deg_parts, x, W1.T, b1.reshape(1, -1))
    a1 = _scatter_hid(h1p, ei4, zeros_h)
    h2p = _tc2(a1, h1p, dis, W2.T, b2.reshape(1, -1))
    a2 = _scatter_cls(h2p, ei4, zeros_c)
    return _tc3(a2, h2p, dis)


# speedup vs baseline: 1.0281x; 1.0281x over previous
import functools

import jax
import jax.numpy as jnp
from jax import lax
from jax.experimental import pallas as pl
from jax.experimental.pallas import tpu as pltpu
from jax.experimental.pallas import tpu_sc as plsc

_N = 10000
_E = 320000
_NFEAT = 128
_NHID = 64
_NCLASS = 40

_NC = 2
_NS = 16
_NT = _NC * _NS
_RPT = _N // _NS
_EPT = _E // _NT
_W = 80
_NWIN = _EPT // _W

_mesh = plsc.VectorSubcoreMesh(core_axis_name="c", subcore_axis_name="s")


_NPAD = 10240
_RPT_PAD = _NPAD // _NS
_LAST = _N - (_NS - 1) * _RPT_PAD
_DEPTH = 16


def _store_out(s, src, dst):
    @pl.when(s < _NS - 1)
    def _full():
        pltpu.sync_copy(src.at[pl.ds(s * _RPT_PAD, _RPT_PAD)],
                        dst.at[pl.ds(s * _RPT_PAD, _RPT_PAD)])

    @pl.when(s == _NS - 1)
    def _clip():
        pltpu.sync_copy(src.at[pl.ds((_NS - 1) * _RPT_PAD, _LAST)],
                        dst.at[pl.ds((_NS - 1) * _RPT_PAD, _LAST)])


@functools.partial(
    pl.kernel,
    out_type=jax.ShapeDtypeStruct((_NC, _NPAD), jnp.float32),
    mesh=_mesh,
    scratch_types=[
        pltpu.VMEM((_NWIN, _W), jnp.int32),
        pltpu.VMEM((_W,), jnp.float32),
        pltpu.VMEM((_RPT_PAD,), jnp.float32),
        pltpu.VMEM_SHARED((_NPAD,), jnp.float32),
        pltpu.SemaphoreType.DMA,
        pltpu.SemaphoreType.DMA,
    ],
)
def _deg_sc(ei_hbm, out_hbm, idxb, ones_v, zb, acc, ssem, isem):
    c = lax.axis_index("c")
    s = lax.axis_index("s")
    tid = c * _NS + s
    one16 = jnp.ones((16,), jnp.float32)
    for i in range(_W // 16):
        ones_v[pl.ds(i * 16, 16)] = one16
    ones_w = ones_v
    ic = pltpu.async_copy(ei_hbm.at[0, tid], idxb, ssem)
    zero16 = jnp.zeros((16,), jnp.float32)

    @pl.loop(0, _RPT_PAD // 16)
    def _zb(i):
        zb[pl.ds(i * 16, 16)] = zero16

    iz = pltpu.async_copy(zb, acc.at[pl.ds(s * _RPT_PAD, _RPT_PAD)], isem)
    ic.wait()
    iz.wait()
    plsc.subcore_barrier()

    @pl.loop(0, _NWIN)
    def _win(w):
        pltpu.async_copy(ones_w, acc.at[idxb.at[w]], ssem, add=True)

        @pl.when(w >= _DEPTH)
        def _drain():
            pltpu.make_async_copy(ones_w, acc.at[idxb.at[w]], ssem).wait()

    for _ in range(_DEPTH):
        pltpu.make_async_copy(ones_w, acc.at[idxb.at[0]], ssem).wait()
    plsc.subcore_barrier()
    pltpu.sync_copy(acc.at[pl.ds(s * _RPT_PAD, _RPT_PAD)],
                    out_hbm.at[c, pl.ds(s * _RPT_PAD, _RPT_PAD)])


_NB = 5
_NG = _NWIN // _NB


def _make_scatter(d):
    @functools.partial(
        pl.kernel,
        out_type=jax.ShapeDtypeStruct((_NC, _N, d), jnp.float32),
        mesh=_mesh,
        scratch_types=[
            pltpu.VMEM((_NWIN, _W), jnp.int32),
            pltpu.VMEM((_NWIN, _W), jnp.int32),
            pltpu.VMEM((_NB, _W, d), jnp.float32),
            pltpu.VMEM((_RPT_PAD // 4, d), jnp.float32),
            pltpu.VMEM_SHARED((_NPAD, d), jnp.float32),
        ] + [pltpu.SemaphoreType.DMA] * (2 * _NB + 1),
        compiler_params=pltpu.CompilerParams(use_tc_tiling_on_sc=False),
    )
    def _scatter_sc(tab_hbm, ei_hbm, out_hbm,
                    ridx, cidx, rows, zb, acc, *sems):
        gsem = sems[:_NB]
        ssem = sems[_NB:2 * _NB]
        isem = sems[2 * _NB]
        c = lax.axis_index("c")
        s = lax.axis_index("s")
        tid = c * _NS + s
        ir = pltpu.async_copy(ei_hbm.at[0, tid], ridx, gsem[0])
        ic = pltpu.async_copy(ei_hbm.at[1, tid], cidx, gsem[1])
        zero16 = jnp.zeros((16,), jnp.float32)

        offs = list(range(0, d - 15, 16))
        if offs[-1] != d - 16:
            offs.append(d - 16)

        @pl.loop(0, _RPT_PAD // 4)
        def _zb(i):
            for j in offs:
                zb[i, pl.ds(j, 16)] = zero16

        for k in range(4):
            pltpu.async_copy(
                zb, acc.at[pl.ds(s * _RPT_PAD + k * (_RPT_PAD // 4),
                                 _RPT_PAD // 4)], isem)
        for k in range(4):
            pltpu.make_async_copy(
                zb, acc.at[pl.ds(s * _RPT_PAD, _RPT_PAD // 4)], isem).wait()
        ir.wait()
        ic.wait()
        plsc.subcore_barrier()

        for b in range(_NB):
            pltpu.async_copy(tab_hbm.at[ridx.at[b]], rows.at[b], gsem[b])

        @pl.loop(0, _NG)
        def _grp(g):
            w0 = g * _NB
            for b in range(_NB):
                pltpu.make_async_copy(
                    tab_hbm.at[ridx.at[w0 + b]], rows.at[b], gsem[b]).wait()
                pltpu.async_copy(
                    rows.at[b], acc.at[cidx.at[w0 + b]], ssem[b], add=True)

            @pl.when(g + 1 < _NG)
            def _next():
                for b in range(_NB):
                    pltpu.make_async_copy(
                        rows.at[b], acc.at[cidx.at[w0 + b]], ssem[b]).wait()
                    pltpu.async_copy(
                        tab_hbm.at[ridx.at[w0 + _NB + b]], rows.at[b], gsem[b])

        w0 = (_NG - 1) * _NB
        for b in range(_NB):
            pltpu.make_async_copy(
                rows.at[b], acc.at[cidx.at[w0 + b]], ssem[b]).wait()
        plsc.subcore_barrier()
        _store_out(s, acc, out_hbm.at[c])

    return _scatter_sc


_scatter_hid = _make_scatter(_NHID)
_scatter_cls = _make_scatter(_NCLASS)


_RB = 1000
_NRB = _N // _RB


def _tc1_body(dp_ref, x_ref, w_ref, b_ref, h_ref, dis_ref):
    deg = 1.0 + dp_ref[0] + dp_ref[1]
    dis = lax.rsqrt(deg)
    h = jnp.dot(x_ref[...], w_ref[...],
                preferred_element_type=jnp.float32) + b_ref[...]
    h_ref[...] = dis * h
    dis_ref[...] = dis


def _tc2_body(a_ref, h_ref, d_ref, w_ref, b_ref, o_ref):
    dis = d_ref[...]
    u = dis * (a_ref[0] + a_ref[1] + h_ref[...])
    z = jnp.maximum(u, 0.0)
    h2 = jnp.dot(z, w_ref[...],
                 preferred_element_type=jnp.float32) + b_ref[...]
    o_ref[...] = dis * h2


def _tc3_body(a_ref, h_ref, d_ref, o_ref):
    u = d_ref[...] * (a_ref[0] + a_ref[1] + h_ref[...])
    m = jnp.max(u, axis=1, keepdims=True)
    lse = jnp.log(jnp.sum(jnp.exp(u - m), axis=1, keepdims=True)) + m
    o_ref[...] = u - lse


def _col_spec(d):
    return pl.BlockSpec((2, _RB, d), lambda i: (0, i, 0))


def _row_spec(d):
    return pl.BlockSpec((_RB, d), lambda i: (i, 0))


def _full_spec(r, c):
    return pl.BlockSpec((r, c), lambda i: (0, 0))


_tc1 = pl.pallas_call(
    _tc1_body,
    grid=(_NRB,),
    in_specs=[_col_spec(1), _row_spec(_NFEAT),
              _full_spec(_NFEAT, _NHID), _full_spec(1, _NHID)],
    out_specs=(_row_spec(_NHID), _row_spec(1)),
    out_shape=(jax.ShapeDtypeStruct((_N, _NHID), jnp.float32),
               jax.ShapeDtypeStruct((_N, 1), jnp.float32)))
_tc2 = pl.pallas_call(
    _tc2_body,
    grid=(_NRB,),
    in_specs=[_col_spec(_NHID), _row_spec(_NHID), _row_spec(1),
              _full_spec(_NHID, _NCLASS), _full_spec(1, _NCLASS)],
    out_specs=_row_spec(_NCLASS),
    out_shape=jax.ShapeDtypeStruct((_N, _NCLASS), jnp.float32))
_tc3 = pl.pallas_call(
    _tc3_body,
    grid=(_NRB,),
    in_specs=[_col_spec(_NCLASS), _row_spec(_NCLASS), _row_spec(1)],
    out_specs=_row_spec(_NCLASS),
    out_shape=jax.ShapeDtypeStruct((_N, _NCLASS), jnp.float32))


@jax.jit
def kernel(x, edge_index, W1, b1, W2, b2):
    ei4 = edge_index.astype(jnp.int32).reshape(2, _NT, _NWIN, _W)

    deg_parts = _deg_sc(ei4)[:, :_N, None]
    h1p, dis = _tc1(deg_parts, x, W1.T, b1.reshape(1, -1))
    a1 = _scatter_hid(h1p, ei4)
    h2p = _tc2(a1, h1p, dis, W2.T, b2.reshape(1, -1))
    a2 = _scatter_cls(h2p, ei4)
    return _tc3(a2, h2p, dis)
